# Initial kernel scaffold; baseline (speedup 1.0000x reference)
import jax
import jax.numpy as jnp
from jax.experimental import pallas as pl


def kernel(x, W1, W2, W3, W4, gc1, bc1, gc2, bc2, gc3, bc3, gc4, bc4, L1w, L1b, g1, b1, L2w, L2b, g2, b2, L3w, L3b):
    raise NotImplementedError("write your pallas kernel here")


# speedup vs baseline: 7.9490x; 7.9490x over previous
import functools

import jax
import jax.numpy as jnp
from jax import lax
from jax.experimental import pallas as pl
from jax.experimental.pallas import tpu as pltpu
from jax.experimental.pallas import tpu_sc as plsc

B, N, K = 8, 2048, 20
BN = B * N
EPS = 1e-5
NEG = -3.0e38


def _knn_uv_body(xt_ref, xa_ref, xxc_ref, xxr_ref, wd_ref, wv_ref, bias_ref,
                 idx_ref, u_ref, v_ref, *, tn):
    b = pl.program_id(0)
    xt = xt_ref[0]
    xa = xa_ref[0]
    dot = lax.dot_general(xt, xa, (((1,), (1,)), ((), ())))
    pd = (2.0 * dot - xxc_ref[0]) - xxr_ref[0]
    iota = lax.broadcasted_iota(jnp.int32, (tn, N), 1)
    base = b * N
    for it in range(K):
        m = jnp.max(pd, axis=1, keepdims=True)
        cand = jnp.where(pd == m, iota, N)
        j = jnp.min(cand, axis=1, keepdims=True)
        idx_ref[0, :, pl.ds(it, 1)] = j + base
        pd = jnp.where(iota == j, NEG, pd)
    u_ref[0] = lax.dot_general(xt, wd_ref[...], (((1,), (0,)), ((), ())))
    v_ref[0] = lax.dot_general(xt, wv_ref[...], (((1,), (0,)), ((), ()))) \
        + bias_ref[0]


def _knn_uv(xp, xx, wd, wv, bias, o, tn=256):
    cp = xp.shape[-1]
    nt = N // tn
    grid = (B, nt)
    return pl.pallas_call(
        functools.partial(_knn_uv_body, tn=tn),
        grid=grid,
        in_specs=[
            pl.BlockSpec((1, tn, cp), lambda b, t: (b, t, 0)),
            pl.BlockSpec((1, N, cp), lambda b, t: (b, 0, 0)),
            pl.BlockSpec((1, tn, 1), lambda b, t: (b, t, 0)),
            pl.BlockSpec((1, 1, N), lambda b, t: (b, 0, 0)),
            pl.BlockSpec((cp, o), lambda b, t: (0, 0)),
            pl.BlockSpec((cp, o), lambda b, t: (0, 0)),
            pl.BlockSpec((1, o), lambda b, t: (0, 0)),
        ],
        out_specs=[
            pl.BlockSpec((1, tn, K), lambda b, t: (b, t, 0)),
            pl.BlockSpec((1, tn, o), lambda b, t: (b, t, 0)),
            pl.BlockSpec((1, tn, o), lambda b, t: (b, t, 0)),
        ],
        out_shape=[
            jax.ShapeDtypeStruct((B, N, K), jnp.int32),
            jax.ShapeDtypeStruct((B, N, o), jnp.float32),
            jax.ShapeDtypeStruct((B, N, o), jnp.float32),
        ],
    )(xp, xp, xx[:, :, None], xx[:, None, :], wd, wv, bias)


_NC, _NS = 2, 16
_NW = _NC * _NS
_PTS = BN // _NW
_P = 32
_NIDX = _P * K // 128
_NCHUNK = _PTS // _P


@functools.lru_cache(maxsize=None)
def _make_gather_max(o):
    mesh = plsc.VectorSubcoreMesh(core_axis_name="c", subcore_axis_name="s")

    @functools.partial(
        pl.kernel,
        mesh=mesh,
        compiler_params=pltpu.CompilerParams(use_tc_tiling_on_sc=False),
        out_type=(
            jax.ShapeDtypeStruct((4, B, o), jnp.float32),
            jax.ShapeDtypeStruct((BN, o), jnp.float32),
        ),
        scratch_types=[
            pltpu.VMEM((_PTS * K // 128, 128), jnp.int32),
            pltpu.VMEM((_P * K, o), jnp.float32),
            pltpu.VMEM((_P, o), jnp.float32),
            pltpu.VMEM((_P, o), jnp.float32),
            pltpu.VMEM((o,), jnp.float32),
            pltpu.SemaphoreType.DMA,
        ],
    )
    def gm(u_hbm, v_hbm, idx_hbm, pmax_hbm, xout_hbm,
           idx_v, rows_v, v_v, xo_v, pm_v, sem):
        cid = lax.axis_index("c")
        sid = lax.axis_index("s")
        wid = cid * _NS + sid
        base = wid * _PTS
        for c in range(o // 16):
            pm_v[pl.ds(c * 16, 16)] = jnp.zeros((16,), jnp.float32)
        nrow = _PTS * K // 128
        pltpu.sync_copy(idx_hbm.at[pl.ds(wid * nrow, nrow)], idx_v)

        def chunk_body(c, carry):
            pbase = base + c * _P
            cps = [
                pltpu.async_copy(u_hbm.at[idx_v.at[c * _NIDX + i]],
                                 rows_v.at[pl.ds(i * 128, 128)], sem)
                for i in range(_NIDX)
            ]
            for cp in cps:
                cp.wait()
            pltpu.sync_copy(v_hbm.at[pl.ds(pbase, _P)], v_v)

            def pt_body(p, carry2):
                r = p * K
                for cc in range(o // 16):
                    sl = pl.ds(cc * 16, 16)
                    m = rows_v[r, sl]
                    for kk in range(1, K):
                        m = jnp.maximum(m, rows_v[r + kk, sl])
                    y = jnp.maximum(m + v_v[p, sl], 0.0)
                    xo_v[p, sl] = y
                    pm_v[sl] = jnp.maximum(pm_v[sl], y)
                return carry2

            lax.fori_loop(0, _P, pt_body, 0)
            pltpu.sync_copy(xo_v, xout_hbm.at[pl.ds(pbase, _P)])
            return carry

        lax.fori_loop(0, _NCHUNK, chunk_body, 0)
        pltpu.sync_copy(pm_v, pmax_hbm.at[wid % 4, wid // 4])

    return gm


def _head_body(p1_ref, p2_ref, p3_ref, p4_ref, w1_ref, b1_ref, w2_ref, b2_ref,
               w3_ref, b3_ref, out_ref):
    h = jnp.concatenate([
        jnp.max(p1_ref[...], axis=0),
        jnp.max(p2_ref[...], axis=0),
        jnp.max(p3_ref[...], axis=0),
        jnp.max(p4_ref[...], axis=0),
    ], axis=1)
    h1 = jax.nn.relu(
        lax.dot_general(h, w1_ref[...], (((1,), (0,)), ((), ()))) + b1_ref[0])
    h2 = jax.nn.relu(
        lax.dot_general(h1, w2_ref[...], (((1,), (0,)), ((), ()))) + b2_ref[0])
    out_ref[...] = lax.dot_general(
        h2, w3_ref[...], (((1,), (0,)), ((), ()))) + b3_ref[0]


def _head(p1, p2, p3, p4, w1, b1, w2, b2, w3, b3):
    full = lambda *s: pl.BlockSpec(s, lambda: tuple(0 for _ in s))
    return pl.pallas_call(
        _head_body,
        in_specs=[
            full(4, B, 64), full(4, B, 64), full(4, B, 64), full(4, B, 128),
            full(320, 1024), full(1, 1024),
            full(1024, 512), full(1, 512),
            full(512, 3), full(1, 3),
        ],
        out_specs=pl.BlockSpec((B, 3), lambda: (0, 0)),
        out_shape=jax.ShapeDtypeStruct((B, 3), jnp.float32),
    )(p1, p2, p3, p4, w1, b1, w2, b2, w3, b3)


def _fold_conv(w, g, bc, cin):
    s = g / jnp.sqrt(1.0 + EPS)
    wd = (w[:, :cin] * s[:, None]).T
    wv = ((w[:, cin:] - w[:, :cin]) * s[:, None]).T
    return wd, wv, bc[None, :]


def _layer(xf, w, g, bc, cin, cout, pad_to=None):
    xx = jnp.sum(xf * xf, axis=2)
    xp = xf
    cp = cin
    if pad_to is not None and pad_to != cin:
        xp = jnp.concatenate(
            [xf, jnp.zeros((B, N, pad_to - cin), jnp.float32)], axis=2)
        cp = pad_to
    wd, wv, bias = _fold_conv(w, g, bc, cin)
    if cp != cin:
        z = jnp.zeros((cp - cin, cout), jnp.float32)
        wd = jnp.concatenate([wd, z], axis=0)
        wv = jnp.concatenate([wv, z], axis=0)
    idx, u, v = _knn_uv(xp, xx, wd, wv, bias, cout)
    idx2d = idx.reshape(BN * K // 128, 128)
    gm = _make_gather_max(cout)
    pmax, xnext = gm(u.reshape(BN, cout), v.reshape(BN, cout), idx2d)
    return xnext, pmax


def kernel(x, W1, W2, W3, W4, gc1, bc1, gc2, bc2, gc3, bc3, gc4, bc4,
           L1w, L1b, g1, b1, L2w, L2b, g2, b2, L3w, L3b):
    x1, pm1 = _layer(x, W1, gc1, bc1, 3, 64, pad_to=8)
    x2, pm2 = _layer(x1.reshape(B, N, 64), W2, gc2, bc2, 64, 64)
    x3, pm3 = _layer(x2.reshape(B, N, 64), W3, gc3, bc3, 64, 64)
    _, pm4 = _layer(x3.reshape(B, N, 64), W4, gc4, bc4, 64, 128)

    s1 = g1 / jnp.sqrt(1.0 + EPS)
    s2 = g2 / jnp.sqrt(1.0 + EPS)
    w1 = L1w.T * s1[None, :]
    bb1 = (L1b * s1 + b1)[None, :]
    w2 = L2w.T * s2[None, :]
    bb2 = (L2b * s2 + b2)[None, :]
    return _head(pm1, pm2, pm3, pm4, w1, bb1, w2, bb2, L3w.T, L3b[None, :])


# speedup vs baseline: 8.6810x; 1.0921x over previous
import functools

import jax
import jax.numpy as jnp
from jax import lax
from jax.experimental import pallas as pl
from jax.experimental.pallas import tpu as pltpu
from jax.experimental.pallas import tpu_sc as plsc

B, N, K = 8, 2048, 20
BN = B * N
EPS = 1e-5
NEG = -3.0e38


def _knn_uv_body(xt_ref, xa_ref, xxc_ref, xxr_ref, wd_ref, wv_ref, bias_ref,
                 idx_ref, u_ref, v_ref, *, tn):
    b = pl.program_id(0)
    xt = xt_ref[0]
    xa = xa_ref[0]
    dot = lax.dot_general(xt, xa, (((1,), (1,)), ((), ())))
    pd = (2.0 * dot - xxc_ref[0]) - xxr_ref[0]
    base = b * N
    h = N // 2
    a = pd[:, :h]
    bb = pd[:, h:]
    iota = lax.broadcasted_iota(jnp.int32, (tn, h), 1)
    ge = a >= bb
    wv = jnp.where(ge, a, bb)
    wi = jnp.where(ge, iota, iota + h)
    lv = jnp.where(ge, bb, a)
    li = jnp.where(ge, iota + h, iota)
    for it in range(K):
        m = jnp.max(wv, axis=1, keepdims=True)
        cand = jnp.where(wv == m, wi, N)
        j = jnp.min(cand, axis=1, keepdims=True)
        idx_ref[0, :, pl.ds(it, 1)] = j + base
        hit = wi == j
        wv = jnp.where(hit, lv, wv)
        wi = jnp.where(hit, li, wi)
        lv = jnp.where(hit, NEG, lv)
    u_ref[0] = lax.dot_general(xt, wd_ref[...], (((1,), (0,)), ((), ())))
    v_ref[0] = lax.dot_general(xt, wv_ref[...], (((1,), (0,)), ((), ()))) \
        + bias_ref[0]


def _knn_uv(xp, xx, wd, wv, bias, o, tn=256):
    cp = xp.shape[-1]
    nt = N // tn
    grid = (B, nt)
    return pl.pallas_call(
        functools.partial(_knn_uv_body, tn=tn),
        grid=grid,
        in_specs=[
            pl.BlockSpec((1, tn, cp), lambda b, t: (b, t, 0)),
            pl.BlockSpec((1, N, cp), lambda b, t: (b, 0, 0)),
            pl.BlockSpec((1, tn, 1), lambda b, t: (b, t, 0)),
            pl.BlockSpec((1, 1, N), lambda b, t: (b, 0, 0)),
            pl.BlockSpec((cp, o), lambda b, t: (0, 0)),
            pl.BlockSpec((cp, o), lambda b, t: (0, 0)),
            pl.BlockSpec((1, o), lambda b, t: (0, 0)),
        ],
        out_specs=[
            pl.BlockSpec((1, tn, K), lambda b, t: (b, t, 0)),
            pl.BlockSpec((1, tn, o), lambda b, t: (b, t, 0)),
            pl.BlockSpec((1, tn, o), lambda b, t: (b, t, 0)),
        ],
        out_shape=[
            jax.ShapeDtypeStruct((B, N, K), jnp.int32),
            jax.ShapeDtypeStruct((B, N, o), jnp.float32),
            jax.ShapeDtypeStruct((B, N, o), jnp.float32),
        ],
    )(xp, xp, xx[:, :, None], xx[:, None, :], wd, wv, bias)


_NC, _NS = 2, 16
_NW = _NC * _NS
_PTS = BN // _NW
_P = 32
_NIDX = _P * K // 128
_NCHUNK = _PTS // _P


@functools.lru_cache(maxsize=None)
def _make_gather_max(o, store_x=True):
    mesh = plsc.VectorSubcoreMesh(core_axis_name="c", subcore_axis_name="s")
    outs = [jax.ShapeDtypeStruct((4, B, o), jnp.float32)]
    if store_x:
        outs.append(jax.ShapeDtypeStruct((BN, o), jnp.float32))

    @functools.partial(
        pl.kernel,
        mesh=mesh,
        compiler_params=pltpu.CompilerParams(use_tc_tiling_on_sc=False),
        out_type=tuple(outs),
        scratch_types=[
            pltpu.VMEM((_PTS * K // 128, 128), jnp.int32),
            pltpu.VMEM((_P * K, o), jnp.float32),
            pltpu.VMEM((_P, o), jnp.float32),
            pltpu.VMEM((_P, o), jnp.float32),
            pltpu.VMEM((o,), jnp.float32),
            pltpu.SemaphoreType.DMA,
        ],
    )
    def gm(u_hbm, v_hbm, idx_hbm, pmax_hbm, *rest):
        if store_x:
            xout_hbm, idx_v, rows_v, v_v, xo_v, pm_v, sem = rest
        else:
            idx_v, rows_v, v_v, xo_v, pm_v, sem = rest
        cid = lax.axis_index("c")
        sid = lax.axis_index("s")
        wid = cid * _NS + sid
        base = wid * _PTS
        for c in range(o // 16):
            pm_v[pl.ds(c * 16, 16)] = jnp.zeros((16,), jnp.float32)
        nrow = _PTS * K // 128
        pltpu.sync_copy(idx_hbm.at[pl.ds(wid * nrow, nrow)], idx_v)

        def chunk_body(c, carry):
            pbase = base + c * _P
            cps = [
                pltpu.async_copy(u_hbm.at[idx_v.at[c * _NIDX + i]],
                                 rows_v.at[pl.ds(i * 128, 128)], sem)
                for i in range(_NIDX)
            ]
            for cp in cps:
                cp.wait()
            pltpu.sync_copy(v_hbm.at[pl.ds(pbase, _P)], v_v)

            def pt_body(p, carry2):
                r = p * K
                for cc in range(o // 16):
                    sl = pl.ds(cc * 16, 16)
                    m = rows_v[r, sl]
                    for kk in range(1, K):
                        m = jnp.maximum(m, rows_v[r + kk, sl])
                    y = jnp.maximum(m + v_v[p, sl], 0.0)
                    xo_v[p, sl] = y
                    pm_v[sl] = jnp.maximum(pm_v[sl], y)
                return carry2

            lax.fori_loop(0, _P, pt_body, 0)
            if store_x:
                pltpu.sync_copy(xo_v, xout_hbm.at[pl.ds(pbase, _P)])
            return carry

        lax.fori_loop(0, _NCHUNK, chunk_body, 0)
        pltpu.sync_copy(pm_v, pmax_hbm.at[wid % 4, wid // 4])

    return gm


def _head_body(p1_ref, p2_ref, p3_ref, p4_ref, w1_ref, b1_ref, w2_ref, b2_ref,
               w3_ref, b3_ref, out_ref):
    h = jnp.concatenate([
        jnp.max(p1_ref[...], axis=0),
        jnp.max(p2_ref[...], axis=0),
        jnp.max(p3_ref[...], axis=0),
        jnp.max(p4_ref[...], axis=0),
    ], axis=1)
    h1 = jax.nn.relu(
        lax.dot_general(h, w1_ref[...], (((1,), (0,)), ((), ()))) + b1_ref[0])
    h2 = jax.nn.relu(
        lax.dot_general(h1, w2_ref[...], (((1,), (0,)), ((), ()))) + b2_ref[0])
    out_ref[...] = lax.dot_general(
        h2, w3_ref[...], (((1,), (0,)), ((), ()))) + b3_ref[0]


def _head(p1, p2, p3, p4, w1, b1, w2, b2, w3, b3):
    full = lambda *s: pl.BlockSpec(s, lambda: tuple(0 for _ in s))
    return pl.pallas_call(
        _head_body,
        in_specs=[
            full(4, B, 64), full(4, B, 64), full(4, B, 64), full(4, B, 128),
            full(320, 1024), full(1, 1024),
            full(1024, 512), full(1, 512),
            full(512, 3), full(1, 3),
        ],
        out_specs=pl.BlockSpec((B, 3), lambda: (0, 0)),
        out_shape=jax.ShapeDtypeStruct((B, 3), jnp.float32),
    )(p1, p2, p3, p4, w1, b1, w2, b2, w3, b3)


def _fold_conv(w, g, bc, cin):
    s = g / jnp.sqrt(1.0 + EPS)
    wd = (w[:, :cin] * s[:, None]).T
    wv = ((w[:, cin:] - w[:, :cin]) * s[:, None]).T
    return wd, wv, bc[None, :]


def _layer(xf, w, g, bc, cin, cout, pad_to=None, store_x=True):
    xx = jnp.sum(xf * xf, axis=2)
    xp = xf
    cp = cin
    if pad_to is not None and pad_to != cin:
        xp = jnp.concatenate(
            [xf, jnp.zeros((B, N, pad_to - cin), jnp.float32)], axis=2)
        cp = pad_to
    wd, wv, bias = _fold_conv(w, g, bc, cin)
    if cp != cin:
        z = jnp.zeros((cp - cin, cout), jnp.float32)
        wd = jnp.concatenate([wd, z], axis=0)
        wv = jnp.concatenate([wv, z], axis=0)
    idx, u, v = _knn_uv(xp, xx, wd, wv, bias, cout)
    idx2d = idx.reshape(BN * K // 128, 128)
    gm = _make_gather_max(cout, store_x)
    res = gm(u.reshape(BN, cout), v.reshape(BN, cout), idx2d)
    if store_x:
        pmax, xnext = res
        return xnext, pmax
    return None, res[0] if isinstance(res, tuple) else res


def kernel(x, W1, W2, W3, W4, gc1, bc1, gc2, bc2, gc3, bc3, gc4, bc4,
           L1w, L1b, g1, b1, L2w, L2b, g2, b2, L3w, L3b):
    x1, pm1 = _layer(x, W1, gc1, bc1, 3, 64, pad_to=8)
    x2, pm2 = _layer(x1.reshape(B, N, 64), W2, gc2, bc2, 64, 64)
    x3, pm3 = _layer(x2.reshape(B, N, 64), W3, gc3, bc3, 64, 64)
    _, pm4 = _layer(x3.reshape(B, N, 64), W4, gc4, bc4, 64, 128, store_x=False)

    s1 = g1 / jnp.sqrt(1.0 + EPS)
    s2 = g2 / jnp.sqrt(1.0 + EPS)
    w1 = L1w.T * s1[None, :]
    bb1 = (L1b * s1 + b1)[None, :]
    w2 = L2w.T * s2[None, :]
    bb2 = (L2b * s2 + b2)[None, :]
    return _head(pm1, pm2, pm3, pm4, w1, bb1, w2, bb2, L3w.T, L3b[None, :])
